# baseline (speedup 1.0000x reference)
import functools

import jax
import jax.numpy as jnp
from jax import lax
from jax.experimental import pallas as pl
from jax.experimental.pallas import tpu as pltpu
from jax.experimental.pallas import tpu_sc as plsc

N_USER = 50000
N_ITEM = 10000
D = 128
E = 300000
OUT = 128

NTILES = 16
CHUNK = 128
CHUNKS_PER_TILE = 147
EDGES_PER_TILE = CHUNKS_PER_TILE * CHUNK
EPAD = NTILES * EDGES_PER_TILE

SLAB = 32
NSLAB = D // SLAB
NBUF = 3

ITEM_PAD = 10240
USER_PAD = 51200
ITEM_CHUNKS = ITEM_PAD // (NTILES * CHUNK)
USER_CHUNKS = USER_PAD // (NTILES * CHUNK)


def _zero_vec(ref, nwords):
    z = jnp.zeros((16,), jnp.float32)

    def body(i, _):
        ref[pl.ds(i * 16, 16)] = z
        return 0

    lax.fori_loop(0, nwords // 16, body, 0)


def _zero_mat(ref, nrows, ncols):
    z = jnp.zeros((16,), jnp.float32)

    def body(i, _):
        ref[i // (ncols // 16), pl.ds((i % (ncols // 16)) * 16, 16)] = z
        return 0

    lax.fori_loop(0, nrows * ncols // 16, body, 0)


def _sc_side(sid, src_ref, dst_ref, table_refs, agg_out, cnt_out,
             row_chunks, agg_sh, cnt_sh, sidx_ring, ridx_ring, didx_ring,
             rowbufs, semsI, semsR, zbuf_v, zrow_v, ones_v):
    nbuf = len(rowbufs)
    ngrp = CHUNKS_PER_TILE // nbuf
    row0 = sid * row_chunks * CHUNK
    ebase = sid * EDGES_PER_TILE

    def sidx_at(b):
        return sidx_ring.at[pl.ds(b * CHUNK, CHUNK)]

    def fire_idx(j, b):
        e = ebase + j * CHUNK
        pltpu.async_copy(src_ref.at[pl.ds(e, CHUNK)], sidx_at(b), semsI[b])
        pltpu.async_copy(dst_ref.at[pl.ds(e, CHUNK)], didx_ring.at[b],
                         semsI[b])

    def wait_idx(j, b):
        e = ebase + j * CHUNK
        pltpu.make_async_copy(src_ref.at[pl.ds(e, CHUNK)], sidx_at(b),
                              semsI[b]).wait()
        pltpu.make_async_copy(dst_ref.at[pl.ds(e, CHUNK)], didx_ring.at[b],
                              semsI[b]).wait()

    def zcnt(i, _):
        pltpu.sync_copy(zrow_v, cnt_sh.at[pl.ds(row0 + i * CHUNK, CHUNK)])
        return 0

    lax.fori_loop(0, row_chunks, zcnt, 0)

    def make_ridx(p, b):
        del p, b

    for p in range(NSLAB):
        def fire_gather(b):
            pltpu.async_copy(table_refs[p].at[sidx_at(b)], rowbufs[b],
                             semsR[b])

        def wait_gather(b):
            pltpu.make_async_copy(table_refs[p].at[sidx_at(b)], rowbufs[b],
                                  semsR[b]).wait()

        def zagg(i, _):
            pltpu.sync_copy(zbuf_v, agg_sh.at[pl.ds(row0 + i * CHUNK, CHUNK)])
            return 0

        lax.fori_loop(0, row_chunks, zagg, 0)
        plsc.subcore_barrier()

        for b in range(nbuf):
            fire_idx(b, b)
        for b in range(nbuf - 1):
            wait_idx(b, b)
            make_ridx(p, b)
            fire_gather(b)

        def egroup(g, _):
            for b in range(nbuf):
                j = g * nbuf + b
                bg = (b + nbuf - 1) % nbuf

                @pl.when((g < ngrp - 1) | (b == 0))
                def _():
                    wait_idx(j + nbuf - 1, bg)
                    make_ridx(p, bg)
                    fire_gather(bg)

                wait_gather(b)
                pltpu.sync_copy(rowbufs[b], agg_sh.at[didx_ring.at[b]],
                                add=True)
                if p == 0:
                    pltpu.sync_copy(ones_v, cnt_sh.at[didx_ring.at[b]],
                                    add=True)

                @pl.when(g < ngrp - 1)
                def _():
                    fire_idx(j + nbuf, b)

            return 0

        lax.fori_loop(0, ngrp, egroup, 0)
        plsc.subcore_barrier()

        aggv = agg_out

        def flush(i, _):
            r = row0 + i * CHUNK
            pltpu.async_copy(agg_sh.at[pl.ds(r, CHUNK)],
                             aggv.at[pl.ds(r, CHUNK), p], semsI[0])
            return 0

        lax.fori_loop(0, row_chunks, flush, 0)

        def flushw(i, _):
            r = row0 + i * CHUNK
            pltpu.make_async_copy(agg_sh.at[pl.ds(r, CHUNK)],
                                  aggv.at[pl.ds(r, CHUNK), p],
                                  semsI[0]).wait()
            return 0

        lax.fori_loop(0, row_chunks, flushw, 0)
        plsc.subcore_barrier()

    def fcnt(i, _):
        r = row0 + i * CHUNK
        pltpu.sync_copy(cnt_sh.at[pl.ds(r, CHUNK)], cnt_out.at[pl.ds(r, CHUNK)])
        return 0

    lax.fori_loop(0, row_chunks, fcnt, 0)


def _sc_aggregate(src_u, dst_i, src_i, dst_u, xu_slabs, xi_slabs):
    mesh = plsc.VectorSubcoreMesh(core_axis_name="c", subcore_axis_name="s")

    @functools.partial(
        pl.kernel,
        out_type=[
            jax.ShapeDtypeStruct((ITEM_PAD, NSLAB, SLAB), jnp.float32),
            jax.ShapeDtypeStruct((ITEM_PAD,), jnp.float32),
            jax.ShapeDtypeStruct((USER_PAD, NSLAB, SLAB), jnp.float32),
            jax.ShapeDtypeStruct((USER_PAD,), jnp.float32),
        ],
        mesh=mesh,
        scratch_types=[
            pltpu.VMEM_SHARED((USER_PAD, SLAB), jnp.float32),
            pltpu.VMEM_SHARED((USER_PAD,), jnp.float32),
            pltpu.VMEM((NBUF * CHUNK,), jnp.int32),
            pltpu.VMEM((NBUF * CHUNK,), jnp.int32),
            pltpu.VMEM((NBUF, CHUNK), jnp.int32),
            pltpu.VMEM((CHUNK, SLAB), jnp.float32),
            pltpu.VMEM((CHUNK, SLAB), jnp.float32),
            pltpu.VMEM((CHUNK, SLAB), jnp.float32),
            pltpu.VMEM((CHUNK, SLAB), jnp.float32),
            pltpu.VMEM((CHUNK,), jnp.float32),
            pltpu.VMEM((CHUNK,), jnp.float32),
            pltpu.SemaphoreType.DMA,
            pltpu.SemaphoreType.DMA,
            pltpu.SemaphoreType.DMA,
            pltpu.SemaphoreType.DMA,
            pltpu.SemaphoreType.DMA,
            pltpu.SemaphoreType.DMA,
        ],
        compiler_params=pltpu.CompilerParams(use_tc_tiling_on_sc=False),
    )
    def sck(srcu_hbm, dsti_hbm, srci_hbm, dstu_hbm,
            xu0, xu1, xu2, xu3, xi0, xi1, xi2, xi3,
            agg_item, cnt_item, agg_user, cnt_user,
            agg_sh, cnt_sh, sidx_ring, ridx_ring, didx_ring, rb0, rb1, rb2,
            zbuf_v, zrow_v, ones_v, semi0, semi1, semi2,
            semr0, semr1, semr2):
        cid = lax.axis_index("c")
        sid = lax.axis_index("s")
        rowbufs = (rb0, rb1, rb2)
        semsI = (semi0, semi1, semi2)
        semsR = (semr0, semr1, semr2)

        _zero_vec(zrow_v, CHUNK)
        _zero_mat(zbuf_v, CHUNK, SLAB)
        one = jnp.ones((16,), jnp.float32)

        def ob(i, _):
            ones_v[pl.ds(i * 16, 16)] = one
            return 0

        lax.fori_loop(0, CHUNK // 16, ob, 0)

        @pl.when(cid == 0)
        def _():
            _sc_side(sid, srcu_hbm, dsti_hbm, (xu0, xu1, xu2, xu3),
                     agg_item, cnt_item, ITEM_CHUNKS,
                     agg_sh, cnt_sh, sidx_ring, ridx_ring, didx_ring,
                     rowbufs, semsI, semsR, zbuf_v, zrow_v, ones_v)

        @pl.when(cid == 1)
        def _():
            _sc_side(sid, srci_hbm, dstu_hbm, (xi0, xi1, xi2, xi3),
                     agg_user, cnt_user, USER_CHUNKS,
                     agg_sh, cnt_sh, sidx_ring, ridx_ring, didx_ring,
                     rowbufs, semsI, semsR, zbuf_v, zrow_v, ones_v)

    return sck(src_u, dst_i, src_i, dst_u, *xu_slabs, *xi_slabs)


_DENSE_R = 512


def _dense_body(agg_ref, cnt_ref, x_ref, wl_ref, bl_ref, wr_ref,
                wlin_ref, blin_ref, out_ref):
    inv = 1.0 / jnp.maximum(cnt_ref[...], 1.0)
    mean = agg_ref[...] * jnp.transpose(inv[None, :])
    t = (jnp.dot(mean, wl_ref[...], preferred_element_type=jnp.float32)
         + bl_ref[...][None, :]
         + jnp.dot(x_ref[...], wr_ref[...], preferred_element_type=jnp.float32))
    z = (jnp.dot(t, wlin_ref[...], preferred_element_type=jnp.float32)
         + blin_ref[...][None, :])
    out_ref[...] = 1.0 / (1.0 + jnp.exp(-z))


def _dense_gate(agg, cnt2d, x, wl, bl, wr, wlin, blin, n_rows):
    R = _DENSE_R
    grid = (n_rows + R - 1) // R
    return pl.pallas_call(
        _dense_body,
        grid=(grid,),
        in_specs=[
            pl.BlockSpec((R, D), lambda i: (i, 0)),
            pl.BlockSpec((R,), lambda i: (i,)),
            pl.BlockSpec((R, D), lambda i: (i, 0)),
            pl.BlockSpec((D, OUT), lambda i: (0, 0)),
            pl.BlockSpec((OUT,), lambda i: (0,)),
            pl.BlockSpec((D, OUT), lambda i: (0, 0)),
            pl.BlockSpec((OUT, OUT), lambda i: (0, 0)),
            pl.BlockSpec((OUT,), lambda i: (0,)),
        ],
        out_specs=pl.BlockSpec((R, OUT), lambda i: (i, 0)),
        out_shape=jax.ShapeDtypeStruct((n_rows, OUT), jnp.float32),
    )(agg, cnt2d, x, wl, bl, wr, wlin, blin)


def kernel(x_user, x_item, h_user, h_item, edge_index_u2i, edge_index_i2u,
           Wl0_u2i, bl0_u2i, Wr0_u2i, Wl0_i2u, bl0_i2u, Wr0_i2u,
           Wl1_u2i, bl1_u2i, Wr1_u2i, Wl1_i2u, bl1_i2u, Wr1_i2u,
           Wlin_user, blin_user, Wlin_item, blin_item):
    npad = EPAD - E
    pad_iota = jnp.arange(npad, dtype=jnp.int32)

    src_u = jnp.concatenate([edge_index_u2i[0].astype(jnp.int32),
                             pad_iota % N_USER])
    dst_i = jnp.concatenate([edge_index_u2i[1].astype(jnp.int32),
                             N_ITEM + pad_iota % (ITEM_PAD - N_ITEM)])
    src_i = jnp.concatenate([edge_index_i2u[0].astype(jnp.int32),
                             pad_iota % N_ITEM])
    dst_u = jnp.concatenate([edge_index_i2u[1].astype(jnp.int32),
                             N_USER + pad_iota % (USER_PAD - N_USER)])

    xu_slabs = [x_user[:, p * SLAB:(p + 1) * SLAB] for p in range(NSLAB)]
    xi_slabs = [x_item[:, p * SLAB:(p + 1) * SLAB] for p in range(NSLAB)]

    agg_item, cnt_item, agg_user, cnt_user = _sc_aggregate(
        src_u, dst_i, src_i, dst_u, xu_slabs, xi_slabs)
    agg_item = agg_item.reshape(ITEM_PAD, D)
    agg_user = agg_user.reshape(USER_PAD, D)

    out_item = _dense_gate(agg_item, cnt_item, x_item,
                           Wl1_u2i, bl1_u2i, Wr1_u2i, Wlin_item, blin_item,
                           N_ITEM)
    out_user = _dense_gate(agg_user, cnt_user, x_user,
                           Wl1_i2u, bl1_i2u, Wr1_i2u, Wlin_user, blin_user,
                           N_USER)
    return (out_user, out_item)


# speedup vs baseline: 1.2930x; 1.2930x over previous
import functools

import jax
import jax.numpy as jnp
from jax import lax
from jax.experimental import pallas as pl
from jax.experimental.pallas import tpu as pltpu
from jax.experimental.pallas import tpu_sc as plsc

N_USER = 50000
N_ITEM = 10000
D = 128
E = 300000
OUT = 128

NTILES = 16
CHUNK = 128
CHUNKS_PER_TILE = 150
EDGES_PER_TILE = CHUNKS_PER_TILE * CHUNK
EPAD = NTILES * EDGES_PER_TILE

SLAB = 32
NSLAB = D // SLAB
NBUF = 3
NIDX = 6

ITEM_PAD = 10240
USER_PAD = 51200
ITEM_CHUNKS = ITEM_PAD // (NTILES * CHUNK)
USER_CHUNKS = USER_PAD // (NTILES * CHUNK)


def _zero_vec(ref, nwords):
    z = jnp.zeros((16,), jnp.float32)

    def body(i, _):
        ref[pl.ds(i * 16, 16)] = z
        return 0

    lax.fori_loop(0, nwords // 16, body, 0)


def _zero_mat(ref, nrows, ncols):
    z = jnp.zeros((16,), jnp.float32)

    def body(i, _):
        ref[i // (ncols // 16), pl.ds((i % (ncols // 16)) * 16, 16)] = z
        return 0

    lax.fori_loop(0, nrows * ncols // 16, body, 0)


def _sc_side(sid, src_ref, dst_ref, table_refs, agg_out, cnt_out,
             row_chunks, agg_sh, cnt_sh, sidx_ring, didx_ring,
             rowbufs, semsI, semsR, semsS, zbuf_v, zrow_v, ones_v):
    nbuf = len(rowbufs)
    ngrp = CHUNKS_PER_TILE // NIDX
    row0 = sid * row_chunks * CHUNK
    ebase = sid * EDGES_PER_TILE

    def sidx_at(s):
        return sidx_ring.at[pl.ds(s * CHUNK, CHUNK)]

    def fire_idx(j, s):
        e = ebase + j * CHUNK
        pltpu.async_copy(src_ref.at[pl.ds(e, CHUNK)], sidx_at(s), semsI[s])
        pltpu.async_copy(dst_ref.at[pl.ds(e, CHUNK)], didx_ring.at[s],
                         semsI[s])

    def wait_idx(j, s):
        e = ebase + j * CHUNK
        pltpu.make_async_copy(src_ref.at[pl.ds(e, CHUNK)], sidx_at(s),
                              semsI[s]).wait()
        pltpu.make_async_copy(dst_ref.at[pl.ds(e, CHUNK)], didx_ring.at[s],
                              semsI[s]).wait()

    def zcnt(i, _):
        pltpu.sync_copy(zrow_v, cnt_sh.at[pl.ds(row0 + i * CHUNK, CHUNK)])
        return 0

    lax.fori_loop(0, row_chunks, zcnt, 0)

    for p in range(NSLAB):
        def fire_gather(s, b):
            pltpu.async_copy(table_refs[p].at[sidx_at(s)], rowbufs[b],
                             semsR[b])

        def wait_gather(s, b):
            pltpu.make_async_copy(table_refs[p].at[sidx_at(s)], rowbufs[b],
                                  semsR[b]).wait()

        def fire_scatter(s, b):
            pltpu.async_copy(rowbufs[b], agg_sh.at[didx_ring.at[s]],
                             semsS[b], add=True)
            if p == 0:
                pltpu.async_copy(ones_v, cnt_sh.at[didx_ring.at[s]],
                                 semsS[b], add=True)

        def wait_scatter(s, b):
            pltpu.make_async_copy(rowbufs[b], agg_sh.at[didx_ring.at[s]],
                                  semsS[b]).wait()
            if p == 0:
                pltpu.make_async_copy(ones_v, cnt_sh.at[didx_ring.at[s]],
                                      semsS[b]).wait()

        def zagg(i, _):
            pltpu.sync_copy(zbuf_v, agg_sh.at[pl.ds(row0 + i * CHUNK, CHUNK)])
            return 0

        lax.fori_loop(0, row_chunks, zagg, 0)
        plsc.subcore_barrier()

        for s in range(NIDX):
            fire_idx(s, s)
        for b in range(nbuf - 1):
            wait_idx(b, b)
            fire_gather(b, b)

        def egroup(g, _):
            for b in range(NIDX):
                j = g * NIDX + b
                br = b % nbuf
                bw = (b + nbuf - 1) % nbuf
                sw = (b + nbuf - 1) % NIDX
                sf = (b + NIDX - 1) % NIDX

                if b == 0:
                    @pl.when(g > 0)
                    def _():
                        wait_scatter((b - 1) % NIDX, (b - 1) % nbuf)
                else:
                    wait_scatter((b - 1) % NIDX, (b - 1) % nbuf)

                if b == 0:
                    @pl.when(g > 0)
                    def _():
                        fire_idx(j + NIDX - 1, sf)
                else:
                    @pl.when(g < ngrp - 1)
                    def _():
                        fire_idx(j + NIDX - 1, sf)

                @pl.when((g < ngrp - 1) | (b + nbuf - 1 < NIDX))
                def _():
                    wait_idx(j + nbuf - 1, sw)
                    fire_gather(sw, bw)

                wait_gather(b % NIDX, br)
                fire_scatter(b, br)

            return 0

        lax.fori_loop(0, ngrp, egroup, 0)
        wait_scatter((CHUNKS_PER_TILE - 1) % NIDX,
                     (CHUNKS_PER_TILE - 1) % nbuf)
        plsc.subcore_barrier()

        aggv = agg_out

        def flush(i, _):
            r = row0 + i * CHUNK
            pltpu.async_copy(agg_sh.at[pl.ds(r, CHUNK)],
                             aggv.at[pl.ds(r, CHUNK), p], semsI[0])
            return 0

        lax.fori_loop(0, row_chunks, flush, 0)

        def flushw(i, _):
            r = row0 + i * CHUNK
            pltpu.make_async_copy(agg_sh.at[pl.ds(r, CHUNK)],
                                  aggv.at[pl.ds(r, CHUNK), p],
                                  semsI[0]).wait()
            return 0

        lax.fori_loop(0, row_chunks, flushw, 0)
        plsc.subcore_barrier()

    def fcnt(i, _):
        r = row0 + i * CHUNK
        pltpu.sync_copy(cnt_sh.at[pl.ds(r, CHUNK)], cnt_out.at[pl.ds(r, CHUNK)])
        return 0

    lax.fori_loop(0, row_chunks, fcnt, 0)


def _sc_aggregate(src_u, dst_i, src_i, dst_u, xu_slabs, xi_slabs):
    mesh = plsc.VectorSubcoreMesh(core_axis_name="c", subcore_axis_name="s")

    @functools.partial(
        pl.kernel,
        out_type=[
            jax.ShapeDtypeStruct((ITEM_PAD, NSLAB, SLAB), jnp.float32),
            jax.ShapeDtypeStruct((ITEM_PAD,), jnp.float32),
            jax.ShapeDtypeStruct((USER_PAD, NSLAB, SLAB), jnp.float32),
            jax.ShapeDtypeStruct((USER_PAD,), jnp.float32),
        ],
        mesh=mesh,
        scratch_types=[
            pltpu.VMEM_SHARED((USER_PAD, SLAB), jnp.float32),
            pltpu.VMEM_SHARED((USER_PAD,), jnp.float32),
            pltpu.VMEM((NIDX * CHUNK,), jnp.int32),
            pltpu.VMEM((NIDX, CHUNK), jnp.int32),
            pltpu.VMEM((CHUNK, SLAB), jnp.float32),
            pltpu.VMEM((CHUNK, SLAB), jnp.float32),
            pltpu.VMEM((CHUNK, SLAB), jnp.float32),
            pltpu.VMEM((CHUNK, SLAB), jnp.float32),
            pltpu.VMEM((CHUNK,), jnp.float32),
            pltpu.VMEM((CHUNK,), jnp.float32),
        ] + [pltpu.SemaphoreType.DMA] * 12,
        compiler_params=pltpu.CompilerParams(use_tc_tiling_on_sc=False),
    )
    def sck(srcu_hbm, dsti_hbm, srci_hbm, dstu_hbm,
            xu0, xu1, xu2, xu3, xi0, xi1, xi2, xi3,
            agg_item, cnt_item, agg_user, cnt_user,
            agg_sh, cnt_sh, sidx_ring, didx_ring, rb0, rb1, rb2,
            zbuf_v, zrow_v, ones_v, *sems):
        cid = lax.axis_index("c")
        sid = lax.axis_index("s")
        rowbufs = (rb0, rb1, rb2)
        semsI = sems[0:6]
        semsR = sems[6:9]
        semsS = sems[9:12]

        _zero_vec(zrow_v, CHUNK)
        _zero_mat(zbuf_v, CHUNK, SLAB)
        one = jnp.ones((16,), jnp.float32)

        def ob(i, _):
            ones_v[pl.ds(i * 16, 16)] = one
            return 0

        lax.fori_loop(0, CHUNK // 16, ob, 0)

        @pl.when(cid == 0)
        def _():
            _sc_side(sid, srcu_hbm, dsti_hbm, (xu0, xu1, xu2, xu3),
                     agg_item, cnt_item, ITEM_CHUNKS,
                     agg_sh, cnt_sh, sidx_ring, didx_ring,
                     rowbufs, semsI, semsR, semsS, zbuf_v, zrow_v, ones_v)

        @pl.when(cid == 1)
        def _():
            _sc_side(sid, srci_hbm, dstu_hbm, (xi0, xi1, xi2, xi3),
                     agg_user, cnt_user, USER_CHUNKS,
                     agg_sh, cnt_sh, sidx_ring, didx_ring,
                     rowbufs, semsI, semsR, semsS, zbuf_v, zrow_v, ones_v)

    return sck(src_u, dst_i, src_i, dst_u, *xu_slabs, *xi_slabs)


_DENSE_R = 512


def _dense_body(agg_ref, cnt_ref, x_ref, wl_ref, bl_ref, wr_ref,
                wlin_ref, blin_ref, out_ref):
    inv = 1.0 / jnp.maximum(cnt_ref[...], 1.0)
    mean = agg_ref[...] * jnp.transpose(inv[None, :])
    t = (jnp.dot(mean, wl_ref[...], preferred_element_type=jnp.float32)
         + bl_ref[...][None, :]
         + jnp.dot(x_ref[...], wr_ref[...], preferred_element_type=jnp.float32))
    z = (jnp.dot(t, wlin_ref[...], preferred_element_type=jnp.float32)
         + blin_ref[...][None, :])
    out_ref[...] = 1.0 / (1.0 + jnp.exp(-z))


def _dense_gate(agg, cnt2d, x, wl, bl, wr, wlin, blin, n_rows):
    R = _DENSE_R
    grid = (n_rows + R - 1) // R
    return pl.pallas_call(
        _dense_body,
        grid=(grid,),
        in_specs=[
            pl.BlockSpec((R, D), lambda i: (i, 0)),
            pl.BlockSpec((R,), lambda i: (i,)),
            pl.BlockSpec((R, D), lambda i: (i, 0)),
            pl.BlockSpec((D, OUT), lambda i: (0, 0)),
            pl.BlockSpec((OUT,), lambda i: (0,)),
            pl.BlockSpec((D, OUT), lambda i: (0, 0)),
            pl.BlockSpec((OUT, OUT), lambda i: (0, 0)),
            pl.BlockSpec((OUT,), lambda i: (0,)),
        ],
        out_specs=pl.BlockSpec((R, OUT), lambda i: (i, 0)),
        out_shape=jax.ShapeDtypeStruct((n_rows, OUT), jnp.float32),
    )(agg, cnt2d, x, wl, bl, wr, wlin, blin)


def kernel(x_user, x_item, h_user, h_item, edge_index_u2i, edge_index_i2u,
           Wl0_u2i, bl0_u2i, Wr0_u2i, Wl0_i2u, bl0_i2u, Wr0_i2u,
           Wl1_u2i, bl1_u2i, Wr1_u2i, Wl1_i2u, bl1_i2u, Wr1_i2u,
           Wlin_user, blin_user, Wlin_item, blin_item):
    npad = EPAD - E
    pad_iota = jnp.arange(npad, dtype=jnp.int32)

    src_u = jnp.concatenate([edge_index_u2i[0].astype(jnp.int32),
                             pad_iota % N_USER])
    dst_i = jnp.concatenate([edge_index_u2i[1].astype(jnp.int32),
                             N_ITEM + pad_iota % (ITEM_PAD - N_ITEM)])
    src_i = jnp.concatenate([edge_index_i2u[0].astype(jnp.int32),
                             pad_iota % N_ITEM])
    dst_u = jnp.concatenate([edge_index_i2u[1].astype(jnp.int32),
                             N_USER + pad_iota % (USER_PAD - N_USER)])

    xu_slabs = [x_user[:, p * SLAB:(p + 1) * SLAB] for p in range(NSLAB)]
    xi_slabs = [x_item[:, p * SLAB:(p + 1) * SLAB] for p in range(NSLAB)]

    agg_item, cnt_item, agg_user, cnt_user = _sc_aggregate(
        src_u, dst_i, src_i, dst_u, xu_slabs, xi_slabs)
    agg_item = agg_item.reshape(ITEM_PAD, D)
    agg_user = agg_user.reshape(USER_PAD, D)

    out_item = _dense_gate(agg_item, cnt_item, x_item,
                           Wl1_u2i, bl1_u2i, Wr1_u2i, Wlin_item, blin_item,
                           N_ITEM)
    out_user = _dense_gate(agg_user, cnt_user, x_user,
                           Wl1_i2u, bl1_i2u, Wr1_i2u, Wlin_user, blin_user,
                           N_USER)
    return (out_user, out_item)


# speedup vs baseline: 1.6601x; 1.2840x over previous
import functools

import jax
import jax.numpy as jnp
from jax import lax
from jax.experimental import pallas as pl
from jax.experimental.pallas import tpu as pltpu
from jax.experimental.pallas import tpu_sc as plsc

N_USER = 50000
N_ITEM = 10000
D = 128
E = 300000
OUT = 128

NTILES = 16
CHUNK = 128
CHUNKS_PER_TILE = 150
EDGES_PER_TILE = CHUNKS_PER_TILE * CHUNK
EPAD = NTILES * EDGES_PER_TILE

SLAB = 32
NSLAB = D // SLAB
NBUF = 3
NIDX = 6

ITEM_PAD = 10240
USER_PAD = 51200
ITEM_CHUNKS = ITEM_PAD // (NTILES * CHUNK)
USER_CHUNKS = USER_PAD // (NTILES * CHUNK)


def _zero_vec(ref, nwords):
    z = jnp.zeros((16,), jnp.float32)

    def body(i, _):
        ref[pl.ds(i * 16, 16)] = z
        return 0

    lax.fori_loop(0, nwords // 16, body, 0)


def _zero_mat(ref, nrows, ncols):
    z = jnp.zeros((16,), jnp.float32)

    def body(i, _):
        ref[i // (ncols // 16), pl.ds((i % (ncols // 16)) * 16, 16)] = z
        return 0

    lax.fori_loop(0, nrows * ncols // 16, body, 0)


def _sc_side(sid, src_ref, dst_ref, table_ref, agg_out, cnt_out,
             row_chunks, agg_sh, cnt_sh, sidx_ring, ridx_ring, didx_ring,
             rowbufs, semsI, semsR, semsS, zbuf_v, zrow_v, ones_v):
    nbuf = len(rowbufs)
    ngrp = CHUNKS_PER_TILE // NIDX
    row0 = sid * row_chunks * CHUNK
    ebase = sid * EDGES_PER_TILE

    def sidx_at(s):
        return sidx_ring.at[pl.ds(s * CHUNK, CHUNK)]

    def fire_idx(j, s):
        e = ebase + j * CHUNK
        pltpu.async_copy(src_ref.at[pl.ds(e, CHUNK)], sidx_at(s), semsI[s])
        pltpu.async_copy(dst_ref.at[pl.ds(e, CHUNK)], didx_ring.at[s],
                         semsI[s])

    def wait_idx(j, s):
        e = ebase + j * CHUNK
        pltpu.make_async_copy(src_ref.at[pl.ds(e, CHUNK)], sidx_at(s),
                              semsI[s]).wait()
        pltpu.make_async_copy(dst_ref.at[pl.ds(e, CHUNK)], didx_ring.at[s],
                              semsI[s]).wait()

    def zcnt(i, _):
        pltpu.sync_copy(zrow_v, cnt_sh.at[pl.ds(row0 + i * CHUNK, CHUNK)])
        return 0

    lax.fori_loop(0, row_chunks, zcnt, 0)

    def ridx_at(s):
        return ridx_ring.at[pl.ds(s * CHUNK, CHUNK)]

    for p in range(NSLAB):
        def make_ridx(s):
            def r16(k, _):
                o = s * CHUNK + k * 16
                ridx_ring[pl.ds(o, 16)] = sidx_ring[pl.ds(o, 16)] * NSLAB + p
                return 0

            lax.fori_loop(0, CHUNK // 16, r16, 0)

        def fire_gather(s, b):
            pltpu.async_copy(table_ref.at[ridx_at(s)], rowbufs[b],
                             semsR[b])

        def wait_gather(s, b):
            pltpu.make_async_copy(table_ref.at[ridx_at(s)], rowbufs[b],
                                  semsR[b]).wait()

        def fire_scatter(s, b):
            pltpu.async_copy(rowbufs[b], agg_sh.at[didx_ring.at[s]],
                             semsS[b], add=True)
            if p == 0:
                pltpu.async_copy(ones_v, cnt_sh.at[didx_ring.at[s]],
                                 semsS[b], add=True)

        def wait_scatter(s, b):
            pltpu.make_async_copy(rowbufs[b], agg_sh.at[didx_ring.at[s]],
                                  semsS[b]).wait()
            if p == 0:
                pltpu.make_async_copy(ones_v, cnt_sh.at[didx_ring.at[s]],
                                      semsS[b]).wait()

        def zagg(i, _):
            pltpu.sync_copy(zbuf_v, agg_sh.at[pl.ds(row0 + i * CHUNK, CHUNK)])
            return 0

        lax.fori_loop(0, row_chunks, zagg, 0)
        plsc.subcore_barrier()

        for s in range(NIDX):
            fire_idx(s, s)
        for b in range(nbuf - 1):
            wait_idx(b, b)
            make_ridx(b)
            fire_gather(b, b)

        def egroup(g, _):
            for b in range(NIDX):
                j = g * NIDX + b
                br = b % nbuf
                bw = (b + nbuf - 1) % nbuf
                sw = (b + nbuf - 1) % NIDX
                sf = (b + NIDX - 1) % NIDX

                if b == 0:
                    @pl.when(g > 0)
                    def _():
                        wait_scatter((b - 1) % NIDX, (b - 1) % nbuf)
                else:
                    wait_scatter((b - 1) % NIDX, (b - 1) % nbuf)

                if b == 0:
                    @pl.when(g > 0)
                    def _():
                        fire_idx(j + NIDX - 1, sf)
                else:
                    @pl.when(g < ngrp - 1)
                    def _():
                        fire_idx(j + NIDX - 1, sf)

                @pl.when((g < ngrp - 1) | (b + nbuf - 1 < NIDX))
                def _():
                    wait_idx(j + nbuf - 1, sw)
                    make_ridx(sw)
                    fire_gather(sw, bw)

                wait_gather(b % NIDX, br)
                fire_scatter(b, br)

            return 0

        lax.fori_loop(0, ngrp, egroup, 0)
        wait_scatter((CHUNKS_PER_TILE - 1) % NIDX,
                     (CHUNKS_PER_TILE - 1) % nbuf)
        plsc.subcore_barrier()

        aggv = agg_out

        def flush(i, _):
            r = row0 + i * CHUNK
            pltpu.async_copy(agg_sh.at[pl.ds(r, CHUNK)],
                             aggv.at[pl.ds(r, CHUNK), p], semsI[0])
            return 0

        lax.fori_loop(0, row_chunks, flush, 0)

        def flushw(i, _):
            r = row0 + i * CHUNK
            pltpu.make_async_copy(agg_sh.at[pl.ds(r, CHUNK)],
                                  aggv.at[pl.ds(r, CHUNK), p],
                                  semsI[0]).wait()
            return 0

        lax.fori_loop(0, row_chunks, flushw, 0)
        plsc.subcore_barrier()

    def fcnt(i, _):
        r = row0 + i * CHUNK
        pltpu.sync_copy(cnt_sh.at[pl.ds(r, CHUNK)], cnt_out.at[pl.ds(r, CHUNK)])
        return 0

    lax.fori_loop(0, row_chunks, fcnt, 0)


def _sc_aggregate(src_u, dst_i, src_i, dst_u, xu_packed, xi_packed):
    mesh = plsc.VectorSubcoreMesh(core_axis_name="c", subcore_axis_name="s")

    @functools.partial(
        pl.kernel,
        out_type=[
            jax.ShapeDtypeStruct((ITEM_PAD, NSLAB, SLAB), jnp.float32),
            jax.ShapeDtypeStruct((ITEM_PAD,), jnp.float32),
            jax.ShapeDtypeStruct((USER_PAD, NSLAB, SLAB), jnp.float32),
            jax.ShapeDtypeStruct((USER_PAD,), jnp.float32),
        ],
        mesh=mesh,
        scratch_types=[
            pltpu.VMEM_SHARED((USER_PAD, SLAB), jnp.float32),
            pltpu.VMEM_SHARED((USER_PAD,), jnp.float32),
            pltpu.VMEM((NIDX * CHUNK,), jnp.int32),
            pltpu.VMEM((NIDX * CHUNK,), jnp.int32),
            pltpu.VMEM((NIDX, CHUNK), jnp.int32),
            pltpu.VMEM((CHUNK, SLAB), jnp.float32),
            pltpu.VMEM((CHUNK, SLAB), jnp.float32),
            pltpu.VMEM((CHUNK, SLAB), jnp.float32),
            pltpu.VMEM((CHUNK, SLAB), jnp.float32),
            pltpu.VMEM((CHUNK,), jnp.float32),
            pltpu.VMEM((CHUNK,), jnp.float32),
        ] + [pltpu.SemaphoreType.DMA] * 12,
        compiler_params=pltpu.CompilerParams(use_tc_tiling_on_sc=False),
    )
    def sck(srcu_hbm, dsti_hbm, srci_hbm, dstu_hbm, xu_hbm, xi_hbm,
            agg_item, cnt_item, agg_user, cnt_user,
            agg_sh, cnt_sh, sidx_ring, ridx_ring, didx_ring, rb0, rb1, rb2,
            zbuf_v, zrow_v, ones_v, *sems):
        cid = lax.axis_index("c")
        sid = lax.axis_index("s")
        rowbufs = (rb0, rb1, rb2)
        semsI = sems[0:6]
        semsR = sems[6:9]
        semsS = sems[9:12]

        _zero_vec(zrow_v, CHUNK)
        _zero_mat(zbuf_v, CHUNK, SLAB)
        one = jnp.ones((16,), jnp.float32)

        def ob(i, _):
            ones_v[pl.ds(i * 16, 16)] = one
            return 0

        lax.fori_loop(0, CHUNK // 16, ob, 0)

        @pl.when(cid == 0)
        def _():
            _sc_side(sid, srcu_hbm, dsti_hbm, xu_hbm,
                     agg_item, cnt_item, ITEM_CHUNKS,
                     agg_sh, cnt_sh, sidx_ring, ridx_ring, didx_ring,
                     rowbufs, semsI, semsR, semsS, zbuf_v, zrow_v, ones_v)

        @pl.when(cid == 1)
        def _():
            _sc_side(sid, srci_hbm, dstu_hbm, xi_hbm,
                     agg_user, cnt_user, USER_CHUNKS,
                     agg_sh, cnt_sh, sidx_ring, ridx_ring, didx_ring,
                     rowbufs, semsI, semsR, semsS, zbuf_v, zrow_v, ones_v)

    return sck(src_u, dst_i, src_i, dst_u, xu_packed, xi_packed)


_DENSE_R = 512


def _dense_body(agg_ref, cnt_ref, x_ref, wl_ref, bl_ref, wr_ref,
                wlin_ref, blin_ref, out_ref):
    inv = 1.0 / jnp.maximum(cnt_ref[...], 1.0)
    mean = agg_ref[...] * jnp.transpose(inv[None, :])
    t = (jnp.dot(mean, wl_ref[...], preferred_element_type=jnp.float32)
         + bl_ref[...][None, :]
         + jnp.dot(x_ref[...], wr_ref[...], preferred_element_type=jnp.float32))
    z = (jnp.dot(t, wlin_ref[...], preferred_element_type=jnp.float32)
         + blin_ref[...][None, :])
    out_ref[...] = 1.0 / (1.0 + jnp.exp(-z))


def _dense_gate(agg, cnt2d, x, wl, bl, wr, wlin, blin, n_rows):
    R = _DENSE_R
    grid = (n_rows + R - 1) // R
    return pl.pallas_call(
        _dense_body,
        grid=(grid,),
        in_specs=[
            pl.BlockSpec((R, D), lambda i: (i, 0)),
            pl.BlockSpec((R,), lambda i: (i,)),
            pl.BlockSpec((R, D), lambda i: (i, 0)),
            pl.BlockSpec((D, OUT), lambda i: (0, 0)),
            pl.BlockSpec((OUT,), lambda i: (0,)),
            pl.BlockSpec((D, OUT), lambda i: (0, 0)),
            pl.BlockSpec((OUT, OUT), lambda i: (0, 0)),
            pl.BlockSpec((OUT,), lambda i: (0,)),
        ],
        out_specs=pl.BlockSpec((R, OUT), lambda i: (i, 0)),
        out_shape=jax.ShapeDtypeStruct((n_rows, OUT), jnp.float32),
    )(agg, cnt2d, x, wl, bl, wr, wlin, blin)


def kernel(x_user, x_item, h_user, h_item, edge_index_u2i, edge_index_i2u,
           Wl0_u2i, bl0_u2i, Wr0_u2i, Wl0_i2u, bl0_i2u, Wr0_i2u,
           Wl1_u2i, bl1_u2i, Wr1_u2i, Wl1_i2u, bl1_i2u, Wr1_i2u,
           Wlin_user, blin_user, Wlin_item, blin_item):
    npad = EPAD - E
    pad_iota = jnp.arange(npad, dtype=jnp.int32)

    src_u = jnp.concatenate([edge_index_u2i[0].astype(jnp.int32),
                             pad_iota % N_USER])
    dst_i = jnp.concatenate([edge_index_u2i[1].astype(jnp.int32),
                             N_ITEM + pad_iota % (ITEM_PAD - N_ITEM)])
    src_i = jnp.concatenate([edge_index_i2u[0].astype(jnp.int32),
                             pad_iota % N_ITEM])
    dst_u = jnp.concatenate([edge_index_i2u[1].astype(jnp.int32),
                             N_USER + pad_iota % (USER_PAD - N_USER)])

    agg_item, cnt_item, agg_user, cnt_user = _sc_aggregate(
        src_u, dst_i, src_i, dst_u,
        x_user.reshape(N_USER * NSLAB, SLAB),
        x_item.reshape(N_ITEM * NSLAB, SLAB))
    agg_item = agg_item.reshape(ITEM_PAD, D)
    agg_user = agg_user.reshape(USER_PAD, D)

    out_item = _dense_gate(agg_item, cnt_item, x_item,
                           Wl1_u2i, bl1_u2i, Wr1_u2i, Wlin_item, blin_item,
                           N_ITEM)
    out_user = _dense_gate(agg_user, cnt_user, x_user,
                           Wl1_i2u, bl1_i2u, Wr1_i2u, Wlin_user, blin_user,
                           N_USER)
    return (out_user, out_item)
